# baseline (speedup 1.0000x reference)
import functools

import jax
import jax.numpy as jnp
from jax import lax
from jax.experimental import pallas as pl
from jax.experimental.pallas import tpu as pltpu
from jax.experimental.pallas import tpu_sc as plsc

_RC = 16
_DEPTH = 3


def kernel(x, pos_table):
    B, S, D = x.shape
    xf = x.reshape(B * S, D)

    info = plsc.get_sparse_core_info()
    NC, NS = info.num_cores, info.num_subcores
    NW = NC * NS
    PW = S // NW
    CPB = PW // _RC
    NCH = B * CPB
    LANES = 16

    mesh = plsc.VectorSubcoreMesh(core_axis_name="c", subcore_axis_name="s")

    @functools.partial(
        pl.kernel,
        out_type=jax.ShapeDtypeStruct((B * S, D), x.dtype),
        mesh=mesh,
        scratch_types=[
            pltpu.VMEM((PW, D), x.dtype),
            pltpu.VMEM((_DEPTH, _RC, D), x.dtype),
            pltpu.SemaphoreType.DMA((_DEPTH,)),
            pltpu.SemaphoreType.DMA((_DEPTH,)),
        ],
    )
    def run(x_hbm, p_hbm, o_hbm, pbuf, xbuf, rsem, wsem):
        w = lax.axis_index("s") * NC + lax.axis_index("c")
        pbase = w * PW
        pltpu.sync_copy(p_hbm.at[pl.ds(pbase, PW)], pbuf)

        def row_base(k):
            return (k // CPB) * S + (k % CPB) * _RC + pbase

        for k in range(min(_DEPTH, NCH)):
            pltpu.make_async_copy(
                x_hbm.at[pl.ds(row_base(k), _RC)], xbuf.at[k % _DEPTH],
                rsem.at[k % _DEPTH],
            ).start()

        for k in range(NCH):
            slot = k % _DEPTH
            pltpu.make_async_copy(
                x_hbm.at[pl.ds(row_base(k), _RC)], xbuf.at[slot], rsem.at[slot]
            ).wait()
            if k >= _DEPTH:
                pltpu.make_async_copy(
                    xbuf.at[slot], o_hbm.at[pl.ds(row_base(k - _DEPTH), _RC)],
                    wsem.at[slot],
                ).wait()

            crow = (k % CPB) * _RC

            def add_row(r, _, slot=slot, crow=crow):
                for j in range(D // LANES):
                    sl = pl.ds(j * LANES, LANES)
                    plsc.addupdate(xbuf.at[slot, r, sl], pbuf[crow + r, sl])
                return 0

            lax.fori_loop(0, _RC, add_row, 0)

            pltpu.make_async_copy(
                xbuf.at[slot], o_hbm.at[pl.ds(row_base(k), _RC)], wsem.at[slot]
            ).start()
            nk = k + _DEPTH
            if nk < NCH:
                pltpu.make_async_copy(
                    x_hbm.at[pl.ds(row_base(nk), _RC)], xbuf.at[slot],
                    rsem.at[slot],
                ).start()

        for k in range(max(NCH - _DEPTH, 0), NCH):
            slot = k % _DEPTH
            pltpu.make_async_copy(
                xbuf.at[slot], o_hbm.at[pl.ds(row_base(k), _RC)], wsem.at[slot]
            ).wait()

    return run(xf, pos_table[:S]).reshape(B, S, D)


# speedup vs baseline: 1.7509x; 1.7509x over previous
import functools

import jax
import jax.numpy as jnp
from jax import lax
from jax.experimental import pallas as pl
from jax.experimental.pallas import tpu as pltpu
from jax.experimental.pallas import tpu_sc as plsc

_RC = 16
_DEPTH = 3
_PREFETCH = 2


def kernel(x, pos_table):
    B, S, D = x.shape
    xf = x.reshape(B * S, D)

    info = plsc.get_sparse_core_info()
    NC, NS = info.num_cores, info.num_subcores
    NW = NC * NS
    PW = S // NW
    CPB = PW // _RC
    NCH = B * CPB
    LANES = 16

    mesh = plsc.VectorSubcoreMesh(core_axis_name="c", subcore_axis_name="s")

    @functools.partial(
        pl.kernel,
        out_type=jax.ShapeDtypeStruct((B * S, D), x.dtype),
        mesh=mesh,
        scratch_types=[
            pltpu.VMEM((PW, D), x.dtype),
            pltpu.VMEM((_DEPTH, _RC, D), x.dtype),
            pltpu.SemaphoreType.DMA,
            pltpu.SemaphoreType.DMA((_DEPTH,)),
            pltpu.SemaphoreType.DMA((_DEPTH,)),
        ],
    )
    def run(x_hbm, p_hbm, o_hbm, pbuf, xbuf, psem, rsem, wsem):
        w = lax.axis_index("s") * NC + lax.axis_index("c")
        pbase = w * PW

        def row_base(k):
            return (k // CPB) * S + (k % CPB) * _RC + pbase

        def read(k):
            return pltpu.make_async_copy(
                x_hbm.at[pl.ds(row_base(k), _RC)], xbuf.at[k % _DEPTH],
                rsem.at[k % _DEPTH],
            )

        def write(k):
            return pltpu.make_async_copy(
                xbuf.at[k % _DEPTH], o_hbm.at[pl.ds(row_base(k), _RC)],
                wsem.at[k % _DEPTH],
            )

        ptab = pltpu.make_async_copy(p_hbm.at[pl.ds(pbase, PW)], pbuf, psem)
        ptab.start()
        for k in range(min(_PREFETCH, NCH)):
            read(k).start()
        ptab.wait()

        waited_writes = set()
        for k in range(NCH):
            slot = k % _DEPTH
            read(k).wait()
            crow = (k % CPB) * _RC

            groups = D // LANES

            @plsc.parallel_loop(0, _RC * groups, unroll=4)
            def add_one(i, slot=slot, crow=crow):
                r = i // groups
                sl = pl.ds((i % groups) * LANES, LANES)
                plsc.addupdate(xbuf.at[slot, r, sl], pbuf[crow + r, sl])

            write(k).start()
            nk = k + _PREFETCH
            if nk < NCH:
                prev = nk - _DEPTH
                if prev >= 0:
                    write(prev).wait()
                    waited_writes.add(prev)
                read(nk).start()

        for k in range(NCH):
            if k not in waited_writes:
                write(k).wait()

    return run(xf, pos_table[:S]).reshape(B, S, D)


# speedup vs baseline: 1.7588x; 1.0045x over previous
import functools

import jax
import jax.numpy as jnp
from jax import lax
from jax.experimental import pallas as pl
from jax.experimental.pallas import tpu as pltpu
from jax.experimental.pallas import tpu_sc as plsc

_RC = 16
_DEPTH = 3
_PREFETCH = 2


def kernel(x, pos_table):
    B, S, D = x.shape
    xf = x.reshape(B * S, D)

    info = plsc.get_sparse_core_info()
    NC, NS = info.num_cores, info.num_subcores
    NW = NC * NS
    PW = S // NW
    CPB = PW // _RC
    NCH = B * CPB
    LANES = 16

    mesh = plsc.VectorSubcoreMesh(core_axis_name="c", subcore_axis_name="s")

    @functools.partial(
        pl.kernel,
        out_type=jax.ShapeDtypeStruct((B * S, D), x.dtype),
        mesh=mesh,
        scratch_types=[
            pltpu.VMEM((PW, D), x.dtype),
            pltpu.VMEM((_DEPTH, _RC, D), x.dtype),
            pltpu.SemaphoreType.DMA,
            pltpu.SemaphoreType.DMA((_DEPTH,)),
            pltpu.SemaphoreType.DMA((_DEPTH,)),
        ],
    )
    def run(x_hbm, p_hbm, o_hbm, pbuf, xbuf, psem, rsem, wsem):
        w = lax.axis_index("s") * NC + lax.axis_index("c")
        pbase = w * PW

        def row_base(k):
            return (k // CPB) * S + (k % CPB) * _RC + pbase

        def read(k):
            return pltpu.make_async_copy(
                x_hbm.at[pl.ds(row_base(k), _RC)], xbuf.at[k % _DEPTH],
                rsem.at[k % _DEPTH],
            )

        def write(k):
            return pltpu.make_async_copy(
                xbuf.at[k % _DEPTH], o_hbm.at[pl.ds(row_base(k), _RC)],
                wsem.at[k % _DEPTH],
            )

        ptab = pltpu.make_async_copy(p_hbm.at[pl.ds(pbase, PW)], pbuf, psem)
        ptab.start()
        for k in range(min(_PREFETCH, NCH)):
            read(k).start()
        ptab.wait()

        waited_writes = set()
        for k in range(NCH):
            slot = k % _DEPTH
            read(k).wait()
            crow = (k % CPB) * _RC

            groups = D // LANES

            @plsc.parallel_loop(0, _RC * groups, unroll=8)
            def add_one(i, slot=slot, crow=crow):
                r = i // groups
                sl = pl.ds((i % groups) * LANES, LANES)
                plsc.addupdate(xbuf.at[slot, r, sl], pbuf[crow + r, sl])

            write(k).start()
            nk = k + _PREFETCH
            if nk < NCH:
                prev = nk - _DEPTH
                if prev >= 0:
                    write(prev).wait()
                    waited_writes.add(prev)
                read(nk).start()

        for k in range(NCH):
            if k not in waited_writes:
                write(k).wait()

    return run(xf, pos_table[:S]).reshape(B, S, D)


# speedup vs baseline: 3.7772x; 2.1477x over previous
import jax
import jax.numpy as jnp
from jax.experimental import pallas as pl
from jax.experimental.pallas import tpu as pltpu


def _add_kernel(x_ref, p_ref, o_ref):
    o_ref[...] = x_ref[...] + p_ref[...]


def kernel(x, pos_table):
    B, S, D = x.shape
    BS = 2048
    grid = (S // BS, B)
    return pl.pallas_call(
        _add_kernel,
        grid=grid,
        in_specs=[
            pl.BlockSpec((1, BS, D), lambda s, b: (b, s, 0)),
            pl.BlockSpec((BS, D), lambda s, b: (s, 0)),
        ],
        out_specs=pl.BlockSpec((1, BS, D), lambda s, b: (b, s, 0)),
        out_shape=jax.ShapeDtypeStruct(x.shape, x.dtype),
        compiler_params=pltpu.CompilerParams(
            dimension_semantics=("parallel", "parallel"),
        ),
    )(x, pos_table[:S])
